# baseline (speedup 1.0000x reference)
import functools

import numpy as np
import jax
import jax.numpy as jnp
from jax import lax
from jax.experimental import pallas as pl
from jax.experimental.pallas import tpu as pltpu
from jax.experimental.pallas import tpu_sc as plsc

_N_LEVELS = 5
_LOG2_T = 17
_T = 1 << _LOG2_T
_BASE_RES = 16
_MAX_RES = 128
_SCALE = float(np.exp((np.log(_MAX_RES) - np.log(_BASE_RES)) / (_N_LEVELS - 1)))
_RES = [int(np.floor(_BASE_RES * (_SCALE ** l))) for l in range(_N_LEVELS)]
_P2 = np.uint32(2654435761)
_P3 = np.uint32(805459861)
_MASK = np.uint32(_T - 1)

_NW = 32
_C = 256
_G = _C // 16
_ER = 10
_SPLIT = (8, 6, 0, 0, 0)


def _enc_body(px_h, py_h, pz_h, tab_h, out_h,
              pxA, pyA, pzA, pxB, pyB, pzB,
              idxA, rowsA, idxB, rowsB, encA, encB, tab_s,
              psemA, psemB, esemA, esemB, *gsems):
    gsA = gsems[:_N_LEVELS]
    gsB = gsems[_N_LEVELS:2 * _N_LEVELS]
    hsA = gsems[2 * _N_LEVELS:3 * _N_LEVELS]
    hsB = gsems[3 * _N_LEVELS:]
    cid = lax.axis_index("c")
    sid = lax.axis_index("s")
    wid = cid * 16 + sid
    n_per_w = px_h.shape[0] // _NW
    n_chunks = n_per_w // _C

    @pl.when(sid == 0)
    def _stage_table():
        pltpu.sync_copy(tab_h, tab_s)
    plsc.subcore_barrier()

    def pos_fire(k, px, py, pz, sem):
        base = wid * n_per_w + k * _C
        pltpu.async_copy(px_h.at[pl.ds(base, _C)], px, sem)
        pltpu.async_copy(py_h.at[pl.ds(base, _C)], py, sem)
        pltpu.async_copy(pz_h.at[pl.ds(base, _C)], pz, sem)

    def pos_wait(px, py, pz, sem):
        pltpu.make_async_copy(px_h.at[pl.ds(0, _C)], px, sem).wait()
        pltpu.make_async_copy(py_h.at[pl.ds(0, _C)], py, sem).wait()
        pltpu.make_async_copy(pz_h.at[pl.ds(0, _C)], pz, sem).wait()

    def idx_pass(px, py, pz, idx):
        def body(g, c):
            o = g * 16
            vx = px[pl.ds(o, 16)]
            vy = py[pl.ds(o, 16)]
            vz = pz[pl.ds(o, 16)]
            for l in range(_N_LEVELS):
                res = jnp.float32(_RES[l])
                x0 = (vx * res).astype(jnp.int32)
                y0 = (vy * res).astype(jnp.int32)
                z0 = (vz * res).astype(jnp.int32)
                lbase = jnp.uint32(l << _LOG2_T)
                hx0 = x0.astype(jnp.uint32) & _MASK
                hx1 = (x0 + 1).astype(jnp.uint32) & _MASK
                hy0 = y0.astype(jnp.uint32) * _P2
                hy1 = (y0 + 1).astype(jnp.uint32) * _P2
                hz0 = z0.astype(jnp.uint32) * _P3
                hz1 = (z0 + 1).astype(jnp.uint32) * _P3
                hyz = (((hy0 ^ hz0) & _MASK) | lbase,
                       ((hy1 ^ hz0) & _MASK) | lbase,
                       ((hy0 ^ hz1) & _MASK) | lbase,
                       ((hy1 ^ hz1) & _MASK) | lbase)
                for k in range(8):
                    hx = hx1 if (k & 1) else hx0
                    idx[pl.ds((l * 8 + k) * _C + o, 16)] = (
                        (hx ^ hyz[k >> 1]).astype(jnp.int32))
            return c
        lax.fori_loop(0, _G, body, 0, unroll=2)

    def gather_fire(idx, rows, sems, hsems):
        for l in range(_N_LEVELS):
            sp = _SPLIT[l]
            if sp > 0:
                sl = pl.ds(l * 8 * _C, sp * _C)
                pltpu.async_copy(tab_h.at[idx.at[sl]], rows.at[sl], hsems[l])
            if sp < 8:
                sl = pl.ds((l * 8 + sp) * _C, (8 - sp) * _C)
                pltpu.async_copy(tab_s.at[idx.at[sl]], rows.at[sl], sems[l])

    def stage(k, px, py, pz, idx, rows, psem, sems, hsems):
        pos_wait(px, py, pz, psem)
        idx_pass(px, py, pz, idx)
        gather_fire(idx, rows, sems, hsems)

    def acc_pass(l, px, py, pz, rows, enc):
        res = jnp.float32(_RES[l])

        def body(g, c):
            o = g * 16
            vx = px[pl.ds(o, 16)]
            vy = py[pl.ds(o, 16)]
            vz = pz[pl.ds(o, 16)]
            xs = vx * res
            ys = vy * res
            zs = vz * res
            wx = xs - xs.astype(jnp.int32).astype(jnp.float32)
            wy = ys - ys.astype(jnp.int32).astype(jnp.float32)
            wz = zs - zs.astype(jnp.int32).astype(jnp.float32)
            f = []
            for k in range(8):
                rv = rows[pl.ds((l * 8 + k) * _C + o, 16)]
                f.append((
                    lax.bitcast_convert_type(lax.shift_left(rv, 16),
                                             jnp.float32),
                    lax.bitcast_convert_type(rv & jnp.int32(-65536),
                                             jnp.float32)))
            for c in range(2):
                g00 = f[0][c] + wx * (f[1][c] - f[0][c])
                g10 = f[2][c] + wx * (f[3][c] - f[2][c])
                g01 = f[4][c] + wx * (f[5][c] - f[4][c])
                g11 = f[6][c] + wx * (f[7][c] - f[6][c])
                h0 = g00 + wy * (g10 - g00)
                h1 = g01 + wy * (g11 - g01)
                enc[2 * l + c, pl.ds(o, 16)] = h0 + wz * (h1 - h0)
            return c
        lax.fori_loop(0, _G, body, 0, unroll=2)

    def drain(k, px, py, pz, idx, rows, enc, esem, sems, hsems, npsem):
        base = wid * n_per_w + k * _C

        @pl.when(k >= 2)
        def _():
            pltpu.make_async_copy(enc, out_h.at[:, pl.ds(0, _C)], esem).wait()

        for l in range(_N_LEVELS):
            sp = _SPLIT[l]
            if sp > 0:
                sl = pl.ds(l * 8 * _C, sp * _C)
                pltpu.make_async_copy(tab_h.at[idx.at[sl]], rows.at[sl],
                                      hsems[l]).wait()
            if sp < 8:
                sl = pl.ds((l * 8 + sp) * _C, (8 - sp) * _C)
                pltpu.make_async_copy(tab_s.at[idx.at[sl]], rows.at[sl],
                                      sems[l]).wait()
            acc_pass(l, px, py, pz, rows, enc)
        pltpu.async_copy(enc, out_h.at[:, pl.ds(base, _C)], esem)

        @pl.when(k + 2 < n_chunks)
        def _():
            pos_fire(k + 2, px, py, pz, npsem)

    pos_fire(0, pxA, pyA, pzA, psemA)
    stage(0, pxA, pyA, pzA, idxA, rowsA, psemA, gsA, hsA)
    pos_fire(1, pxB, pyB, pzB, psemB)

    def body2(j, carry):
        ka = 2 * j
        kb = 2 * j + 1
        stage(kb, pxB, pyB, pzB, idxB, rowsB, psemB, gsB, hsB)
        drain(ka, pxA, pyA, pzA, idxA, rowsA, encA, esemA, gsA, hsA, psemA)

        @pl.when(ka + 2 < n_chunks)
        def _():
            stage(ka + 2, pxA, pyA, pzA, idxA, rowsA, psemA, gsA, hsA)
        drain(kb, pxB, pyB, pzB, idxB, rowsB, encB, esemB, gsB, hsB, psemB)
        return carry

    lax.fori_loop(0, n_chunks // 2, body2, 0, unroll=False)

    pltpu.make_async_copy(encA, out_h.at[:, pl.ds(0, _C)], esemA).wait()
    pltpu.make_async_copy(encB, out_h.at[:, pl.ds(0, _C)], esemB).wait()


def _encode(px, py, pz, tabf):
    n = px.shape[0]
    mesh = plsc.VectorSubcoreMesh(core_axis_name="c", subcore_axis_name="s")
    f = pl.kernel(
        _enc_body,
        mesh=mesh,
        out_type=jax.ShapeDtypeStruct((_ER, n), jnp.float32),
        scratch_types=(
            [pltpu.VMEM((_C,), jnp.float32) for _ in range(6)]
            + [
                pltpu.VMEM((_N_LEVELS * 8 * _C,), jnp.int32),
                pltpu.VMEM((_N_LEVELS * 8 * _C,), jnp.int32),
                pltpu.VMEM((_N_LEVELS * 8 * _C,), jnp.int32),
                pltpu.VMEM((_N_LEVELS * 8 * _C,), jnp.int32),
                pltpu.VMEM((_ER, _C), jnp.float32),
                pltpu.VMEM((_ER, _C), jnp.float32),
                pltpu.VMEM_SHARED((_N_LEVELS * _T,), jnp.int32),
            ]
            + [pltpu.SemaphoreType.DMA for _ in range(4 + 4 * _N_LEVELS)]
        ),
    )
    return f(px, py, pz, tabf)


def _mlp_body(enc_ref, pt_ref, w1_ref, w2_ref, o_ref):
    x = enc_ref[...]
    xb = x.astype(jnp.bfloat16)
    h = lax.dot_general(w1_ref[...].astype(jnp.bfloat16), xb,
                        (((0,), (0,)), ((), ())),
                        preferred_element_type=jnp.float32)
    h = jnp.maximum(h, 0.0).astype(jnp.bfloat16)
    d = lax.dot_general(w2_ref[...].astype(jnp.bfloat16), h,
                        (((0,), (0,)), ((), ())),
                        preferred_element_type=jnp.float32)
    p = pt_ref[...]
    ok = jnp.all((p > 0.0) & (p < 1.0), axis=0, keepdims=True)
    o_ref[...] = jnp.where(ok, jnp.exp(d - 1.0), jnp.float32(0.0))


def _mlp(enc, pt, w1, w2):
    n = enc.shape[1]
    nb = 8192
    return pl.pallas_call(
        _mlp_body,
        grid=(n // nb,),
        in_specs=[
            pl.BlockSpec((_ER, nb), lambda i: (0, i)),
            pl.BlockSpec((3, nb), lambda i: (0, i)),
            pl.BlockSpec((_ER, 64), lambda i: (0, 0)),
            pl.BlockSpec((64, 1), lambda i: (0, 0)),
        ],
        out_specs=pl.BlockSpec((1, nb), lambda i: (0, i)),
        out_shape=jax.ShapeDtypeStruct((1, n), jnp.float32),
    )(enc, pt, w1, w2)


def kernel(positions, table, W1, W2):
    n = positions.shape[0]
    tabf = jax.lax.bitcast_convert_type(
        table.reshape(_N_LEVELS * _T, 2).astype(jnp.bfloat16), jnp.int32)
    pt = positions.T
    px, py, pz = pt[0], pt[1], pt[2]
    enc = _encode(px, py, pz, tabf)
    out = _mlp(enc, pt, W1, W2)
    return out.reshape(n, 1)


# speedup vs baseline: 1.4180x; 1.4180x over previous
import functools

import numpy as np
import jax
import jax.numpy as jnp
from jax import lax
from jax.experimental import pallas as pl
from jax.experimental.pallas import tpu as pltpu
from jax.experimental.pallas import tpu_sc as plsc

_N_LEVELS = 5
_LOG2_T = 17
_T = 1 << _LOG2_T
_BASE_RES = 16
_MAX_RES = 128
_SCALE = float(np.exp((np.log(_MAX_RES) - np.log(_BASE_RES)) / (_N_LEVELS - 1)))
_RES = [int(np.floor(_BASE_RES * (_SCALE ** l))) for l in range(_N_LEVELS)]
_P2 = np.uint32(2654435761)
_P3 = np.uint32(805459861)
_MASK = np.uint32(_T - 1)

_NW = 32
_C = 256
_G = _C // 16
_ER = 10
_SPLIT = (6, 0, 0, 0, 0)


def _enc_body(px_h, py_h, pz_h, tab_h, out_h,
              pxA, pyA, pzA, pxB, pyB, pzB,
              idxA, rowsA, idxB, rowsB, encA, encB, tab_s,
              psemA, psemB, esemA, esemB, *gsems):
    gsA = gsems[:_N_LEVELS]
    gsB = gsems[_N_LEVELS:2 * _N_LEVELS]
    hsA = gsems[2 * _N_LEVELS:3 * _N_LEVELS]
    hsB = gsems[3 * _N_LEVELS:]
    cid = lax.axis_index("c")
    sid = lax.axis_index("s")
    wid = cid * 16 + sid
    n_per_w = px_h.shape[0] // _NW
    n_chunks = n_per_w // _C

    @pl.when(sid == 0)
    def _stage_table():
        pltpu.sync_copy(tab_h, tab_s)
    plsc.subcore_barrier()

    def pos_fire(k, px, py, pz, sem):
        base = wid * n_per_w + k * _C
        pltpu.async_copy(px_h.at[pl.ds(base, _C)], px, sem)
        pltpu.async_copy(py_h.at[pl.ds(base, _C)], py, sem)
        pltpu.async_copy(pz_h.at[pl.ds(base, _C)], pz, sem)

    def pos_wait(px, py, pz, sem):
        pltpu.make_async_copy(px_h.at[pl.ds(0, _C)], px, sem).wait()
        pltpu.make_async_copy(py_h.at[pl.ds(0, _C)], py, sem).wait()
        pltpu.make_async_copy(pz_h.at[pl.ds(0, _C)], pz, sem).wait()

    def idx_pass(px, py, pz, idx):
        def body(g, c):
            o = g * 16
            vx = px[pl.ds(o, 16)]
            vy = py[pl.ds(o, 16)]
            vz = pz[pl.ds(o, 16)]
            for l in range(_N_LEVELS):
                res = jnp.float32(_RES[l])
                x0 = (vx * res).astype(jnp.int32)
                y0 = (vy * res).astype(jnp.int32)
                z0 = (vz * res).astype(jnp.int32)
                lbase = jnp.uint32(l << _LOG2_T)
                hx0 = x0.astype(jnp.uint32) & _MASK
                hx1 = (x0 + 1).astype(jnp.uint32) & _MASK
                hy0 = y0.astype(jnp.uint32) * _P2
                hy1 = (y0 + 1).astype(jnp.uint32) * _P2
                hz0 = z0.astype(jnp.uint32) * _P3
                hz1 = (z0 + 1).astype(jnp.uint32) * _P3
                hyz = (((hy0 ^ hz0) & _MASK) | lbase,
                       ((hy1 ^ hz0) & _MASK) | lbase,
                       ((hy0 ^ hz1) & _MASK) | lbase,
                       ((hy1 ^ hz1) & _MASK) | lbase)
                for k in range(8):
                    hx = hx1 if (k & 1) else hx0
                    idx[pl.ds((l * 8 + k) * _C + o, 16)] = (
                        (hx ^ hyz[k >> 1]).astype(jnp.int32))
            return c
        lax.fori_loop(0, _G, body, 0, unroll=2)

    def gather_fire(idx, rows, sems, hsems):
        for l in range(_N_LEVELS):
            sp = _SPLIT[l]
            if sp > 0:
                sl = pl.ds(l * 8 * _C, sp * _C)
                pltpu.async_copy(tab_h.at[idx.at[sl]], rows.at[sl], hsems[l])
            if sp < 8:
                sl = pl.ds((l * 8 + sp) * _C, (8 - sp) * _C)
                pltpu.async_copy(tab_s.at[idx.at[sl]], rows.at[sl], sems[l])

    def stage(k, px, py, pz, idx, rows, psem, sems, hsems):
        pos_wait(px, py, pz, psem)
        idx_pass(px, py, pz, idx)
        gather_fire(idx, rows, sems, hsems)

    def acc_pass(l, px, py, pz, rows, enc):
        res = jnp.float32(_RES[l])

        def body(g, c):
            o = g * 16
            vx = px[pl.ds(o, 16)]
            vy = py[pl.ds(o, 16)]
            vz = pz[pl.ds(o, 16)]
            xs = vx * res
            ys = vy * res
            zs = vz * res
            wx = xs - xs.astype(jnp.int32).astype(jnp.float32)
            wy = ys - ys.astype(jnp.int32).astype(jnp.float32)
            wz = zs - zs.astype(jnp.int32).astype(jnp.float32)
            f = []
            for k in range(8):
                rv = rows[pl.ds((l * 8 + k) * _C + o, 16)]
                f.append((
                    lax.bitcast_convert_type(lax.shift_left(rv, 16),
                                             jnp.float32),
                    lax.bitcast_convert_type(rv & jnp.int32(-65536),
                                             jnp.float32)))
            for c in range(2):
                g00 = f[0][c] + wx * (f[1][c] - f[0][c])
                g10 = f[2][c] + wx * (f[3][c] - f[2][c])
                g01 = f[4][c] + wx * (f[5][c] - f[4][c])
                g11 = f[6][c] + wx * (f[7][c] - f[6][c])
                h0 = g00 + wy * (g10 - g00)
                h1 = g01 + wy * (g11 - g01)
                enc[2 * l + c, pl.ds(o, 16)] = h0 + wz * (h1 - h0)
            return c
        lax.fori_loop(0, _G, body, 0, unroll=2)

    def drain(k, px, py, pz, idx, rows, enc, esem, sems, hsems, npsem):
        base = wid * n_per_w + k * _C

        @pl.when(k >= 2)
        def _():
            pltpu.make_async_copy(enc, out_h.at[:, pl.ds(0, _C)], esem).wait()

        for l in range(_N_LEVELS):
            sp = _SPLIT[l]
            if sp > 0:
                sl = pl.ds(l * 8 * _C, sp * _C)
                pltpu.make_async_copy(tab_h.at[idx.at[sl]], rows.at[sl],
                                      hsems[l]).wait()
            if sp < 8:
                sl = pl.ds((l * 8 + sp) * _C, (8 - sp) * _C)
                pltpu.make_async_copy(tab_s.at[idx.at[sl]], rows.at[sl],
                                      sems[l]).wait()
            acc_pass(l, px, py, pz, rows, enc)
        pltpu.async_copy(enc, out_h.at[:, pl.ds(base, _C)], esem)

        @pl.when(k + 2 < n_chunks)
        def _():
            pos_fire(k + 2, px, py, pz, npsem)

    pos_fire(0, pxA, pyA, pzA, psemA)
    stage(0, pxA, pyA, pzA, idxA, rowsA, psemA, gsA, hsA)
    pos_fire(1, pxB, pyB, pzB, psemB)

    def body2(j, carry):
        ka = 2 * j
        kb = 2 * j + 1
        stage(kb, pxB, pyB, pzB, idxB, rowsB, psemB, gsB, hsB)
        drain(ka, pxA, pyA, pzA, idxA, rowsA, encA, esemA, gsA, hsA, psemA)

        @pl.when(ka + 2 < n_chunks)
        def _():
            stage(ka + 2, pxA, pyA, pzA, idxA, rowsA, psemA, gsA, hsA)
        drain(kb, pxB, pyB, pzB, idxB, rowsB, encB, esemB, gsB, hsB, psemB)
        return carry

    lax.fori_loop(0, n_chunks // 2, body2, 0, unroll=False)

    pltpu.make_async_copy(encA, out_h.at[:, pl.ds(0, _C)], esemA).wait()
    pltpu.make_async_copy(encB, out_h.at[:, pl.ds(0, _C)], esemB).wait()


def _encode(px, py, pz, tabf):
    n = px.shape[0]
    mesh = plsc.VectorSubcoreMesh(core_axis_name="c", subcore_axis_name="s")
    f = pl.kernel(
        _enc_body,
        mesh=mesh,
        out_type=jax.ShapeDtypeStruct((_ER, n), jnp.float32),
        scratch_types=(
            [pltpu.VMEM((_C,), jnp.float32) for _ in range(6)]
            + [
                pltpu.VMEM((_N_LEVELS * 8 * _C,), jnp.int32),
                pltpu.VMEM((_N_LEVELS * 8 * _C,), jnp.int32),
                pltpu.VMEM((_N_LEVELS * 8 * _C,), jnp.int32),
                pltpu.VMEM((_N_LEVELS * 8 * _C,), jnp.int32),
                pltpu.VMEM((_ER, _C), jnp.float32),
                pltpu.VMEM((_ER, _C), jnp.float32),
                pltpu.VMEM_SHARED((_N_LEVELS * _T,), jnp.int32),
            ]
            + [pltpu.SemaphoreType.DMA for _ in range(4 + 4 * _N_LEVELS)]
        ),
    )
    return f(px, py, pz, tabf)


def _mlp_body(enc_ref, pt_ref, w1_ref, w2_ref, o_ref):
    x = enc_ref[...]
    xb = x.astype(jnp.bfloat16)
    h = lax.dot_general(w1_ref[...].astype(jnp.bfloat16), xb,
                        (((0,), (0,)), ((), ())),
                        preferred_element_type=jnp.float32)
    h = jnp.maximum(h, 0.0).astype(jnp.bfloat16)
    d = lax.dot_general(w2_ref[...].astype(jnp.bfloat16), h,
                        (((0,), (0,)), ((), ())),
                        preferred_element_type=jnp.float32)
    p = pt_ref[...]
    ok = jnp.all((p > 0.0) & (p < 1.0), axis=0, keepdims=True)
    o_ref[...] = jnp.where(ok, jnp.exp(d - 1.0), jnp.float32(0.0))


def _mlp(enc, pt, w1, w2):
    n = enc.shape[1]
    nb = 8192
    return pl.pallas_call(
        _mlp_body,
        grid=(n // nb,),
        in_specs=[
            pl.BlockSpec((_ER, nb), lambda i: (0, i)),
            pl.BlockSpec((3, nb), lambda i: (0, i)),
            pl.BlockSpec((_ER, 64), lambda i: (0, 0)),
            pl.BlockSpec((64, 1), lambda i: (0, 0)),
        ],
        out_specs=pl.BlockSpec((1, nb), lambda i: (0, i)),
        out_shape=jax.ShapeDtypeStruct((1, n), jnp.float32),
    )(enc, pt, w1, w2)


def kernel(positions, table, W1, W2):
    n = positions.shape[0]
    tabf = jax.lax.bitcast_convert_type(
        table.reshape(_N_LEVELS * _T, 2).astype(jnp.bfloat16), jnp.int32)
    pt = positions.T
    px, py, pz = pt[0], pt[1], pt[2]
    enc = _encode(px, py, pz, tabf)
    out = _mlp(enc, pt, W1, W2)
    return out.reshape(n, 1)


# speedup vs baseline: 1.7457x; 1.2311x over previous
import functools

import numpy as np
import jax
import jax.numpy as jnp
from jax import lax
from jax.experimental import pallas as pl
from jax.experimental.pallas import tpu as pltpu
from jax.experimental.pallas import tpu_sc as plsc

_N_LEVELS = 5
_LOG2_T = 17
_T = 1 << _LOG2_T
_BASE_RES = 16
_MAX_RES = 128
_SCALE = float(np.exp((np.log(_MAX_RES) - np.log(_BASE_RES)) / (_N_LEVELS - 1)))
_RES = [int(np.floor(_BASE_RES * (_SCALE ** l))) for l in range(_N_LEVELS)]
_P2 = np.uint32(2654435761)
_P3 = np.uint32(805459861)
_MASK = np.uint32(_T - 1)

_NW = 32
_C = 256
_G = _C // 16
_ER = 10
_SPLIT = (0, 0, 0, 0, 0)


def _enc_body(px_h, py_h, pz_h, tab_h, out_h,
              pxA, pyA, pzA, pxB, pyB, pzB,
              idxA, rowsA, idxB, rowsB, encA, encB, tab_s,
              psemA, psemB, esemA, esemB, *gsems):
    gsA = gsems[:_N_LEVELS]
    gsB = gsems[_N_LEVELS:2 * _N_LEVELS]
    hsA = gsems[2 * _N_LEVELS:3 * _N_LEVELS]
    hsB = gsems[3 * _N_LEVELS:]
    cid = lax.axis_index("c")
    sid = lax.axis_index("s")
    wid = cid * 16 + sid
    n_per_w = px_h.shape[0] // _NW
    n_chunks = n_per_w // _C

    @pl.when(sid == 0)
    def _stage_table():
        pltpu.sync_copy(tab_h, tab_s)
    plsc.subcore_barrier()

    def pos_fire(k, px, py, pz, sem):
        base = wid * n_per_w + k * _C
        pltpu.async_copy(px_h.at[pl.ds(base, _C)], px, sem)
        pltpu.async_copy(py_h.at[pl.ds(base, _C)], py, sem)
        pltpu.async_copy(pz_h.at[pl.ds(base, _C)], pz, sem)

    def pos_wait(px, py, pz, sem):
        pltpu.make_async_copy(px_h.at[pl.ds(0, _C)], px, sem).wait()
        pltpu.make_async_copy(py_h.at[pl.ds(0, _C)], py, sem).wait()
        pltpu.make_async_copy(pz_h.at[pl.ds(0, _C)], pz, sem).wait()

    def idx_pass(px, py, pz, idx):
        def body(g, c):
            o = g * 16
            vx = px[pl.ds(o, 16)]
            vy = py[pl.ds(o, 16)]
            vz = pz[pl.ds(o, 16)]
            for l in range(_N_LEVELS):
                res = jnp.float32(_RES[l])
                x0 = (vx * res).astype(jnp.int32)
                y0 = (vy * res).astype(jnp.int32)
                z0 = (vz * res).astype(jnp.int32)
                lbase = jnp.uint32(l << _LOG2_T)
                hx0 = x0.astype(jnp.uint32) & _MASK
                hx1 = (x0 + 1).astype(jnp.uint32) & _MASK
                hy0 = y0.astype(jnp.uint32) * _P2
                hy1 = (y0 + 1).astype(jnp.uint32) * _P2
                hz0 = z0.astype(jnp.uint32) * _P3
                hz1 = (z0 + 1).astype(jnp.uint32) * _P3
                hyz = (((hy0 ^ hz0) & _MASK) | lbase,
                       ((hy1 ^ hz0) & _MASK) | lbase,
                       ((hy0 ^ hz1) & _MASK) | lbase,
                       ((hy1 ^ hz1) & _MASK) | lbase)
                for k in range(8):
                    hx = hx1 if (k & 1) else hx0
                    idx[pl.ds((l * 8 + k) * _C + o, 16)] = (
                        (hx ^ hyz[k >> 1]).astype(jnp.int32))
            return c
        lax.fori_loop(0, _G, body, 0, unroll=2)

    def gather_fire(idx, rows, sems, hsems):
        for l in range(_N_LEVELS):
            sp = _SPLIT[l]
            if sp > 0:
                sl = pl.ds(l * 8 * _C, sp * _C)
                pltpu.async_copy(tab_h.at[idx.at[sl]], rows.at[sl], hsems[l])
            if sp < 8:
                sl = pl.ds((l * 8 + sp) * _C, (8 - sp) * _C)
                pltpu.async_copy(tab_s.at[idx.at[sl]], rows.at[sl], sems[l])

    def stage(k, px, py, pz, idx, rows, psem, sems, hsems):
        pos_wait(px, py, pz, psem)
        idx_pass(px, py, pz, idx)
        gather_fire(idx, rows, sems, hsems)

    def acc_pass(l, px, py, pz, rows, enc):
        res = jnp.float32(_RES[l])

        def body(g, c):
            o = g * 16
            vx = px[pl.ds(o, 16)]
            vy = py[pl.ds(o, 16)]
            vz = pz[pl.ds(o, 16)]
            xs = vx * res
            ys = vy * res
            zs = vz * res
            wx = xs - xs.astype(jnp.int32).astype(jnp.float32)
            wy = ys - ys.astype(jnp.int32).astype(jnp.float32)
            wz = zs - zs.astype(jnp.int32).astype(jnp.float32)
            f = []
            for k in range(8):
                rv = rows[pl.ds((l * 8 + k) * _C + o, 16)]
                f.append((
                    lax.bitcast_convert_type(lax.shift_left(rv, 16),
                                             jnp.float32),
                    lax.bitcast_convert_type(rv & jnp.int32(-65536),
                                             jnp.float32)))
            for c in range(2):
                g00 = f[0][c] + wx * (f[1][c] - f[0][c])
                g10 = f[2][c] + wx * (f[3][c] - f[2][c])
                g01 = f[4][c] + wx * (f[5][c] - f[4][c])
                g11 = f[6][c] + wx * (f[7][c] - f[6][c])
                h0 = g00 + wy * (g10 - g00)
                h1 = g01 + wy * (g11 - g01)
                enc[2 * l + c, pl.ds(o, 16)] = h0 + wz * (h1 - h0)
            return c
        lax.fori_loop(0, _G, body, 0, unroll=2)

    def drain(k, px, py, pz, idx, rows, enc, esem, sems, hsems, npsem):
        base = wid * n_per_w + k * _C

        @pl.when(k >= 2)
        def _():
            pltpu.make_async_copy(enc, out_h.at[:, pl.ds(0, _C)], esem).wait()

        for l in range(_N_LEVELS):
            sp = _SPLIT[l]
            if sp > 0:
                sl = pl.ds(l * 8 * _C, sp * _C)
                pltpu.make_async_copy(tab_h.at[idx.at[sl]], rows.at[sl],
                                      hsems[l]).wait()
            if sp < 8:
                sl = pl.ds((l * 8 + sp) * _C, (8 - sp) * _C)
                pltpu.make_async_copy(tab_s.at[idx.at[sl]], rows.at[sl],
                                      sems[l]).wait()
            acc_pass(l, px, py, pz, rows, enc)
        pltpu.async_copy(enc, out_h.at[:, pl.ds(base, _C)], esem)

        @pl.when(k + 2 < n_chunks)
        def _():
            pos_fire(k + 2, px, py, pz, npsem)

    pos_fire(0, pxA, pyA, pzA, psemA)
    stage(0, pxA, pyA, pzA, idxA, rowsA, psemA, gsA, hsA)
    pos_fire(1, pxB, pyB, pzB, psemB)

    def body2(j, carry):
        ka = 2 * j
        kb = 2 * j + 1
        stage(kb, pxB, pyB, pzB, idxB, rowsB, psemB, gsB, hsB)
        drain(ka, pxA, pyA, pzA, idxA, rowsA, encA, esemA, gsA, hsA, psemA)

        @pl.when(ka + 2 < n_chunks)
        def _():
            stage(ka + 2, pxA, pyA, pzA, idxA, rowsA, psemA, gsA, hsA)
        drain(kb, pxB, pyB, pzB, idxB, rowsB, encB, esemB, gsB, hsB, psemB)
        return carry

    lax.fori_loop(0, n_chunks // 2, body2, 0, unroll=False)

    pltpu.make_async_copy(encA, out_h.at[:, pl.ds(0, _C)], esemA).wait()
    pltpu.make_async_copy(encB, out_h.at[:, pl.ds(0, _C)], esemB).wait()


def _encode(px, py, pz, tabf):
    n = px.shape[0]
    mesh = plsc.VectorSubcoreMesh(core_axis_name="c", subcore_axis_name="s")
    f = pl.kernel(
        _enc_body,
        mesh=mesh,
        out_type=jax.ShapeDtypeStruct((_ER, n), jnp.float32),
        scratch_types=(
            [pltpu.VMEM((_C,), jnp.float32) for _ in range(6)]
            + [
                pltpu.VMEM((_N_LEVELS * 8 * _C,), jnp.int32),
                pltpu.VMEM((_N_LEVELS * 8 * _C,), jnp.int32),
                pltpu.VMEM((_N_LEVELS * 8 * _C,), jnp.int32),
                pltpu.VMEM((_N_LEVELS * 8 * _C,), jnp.int32),
                pltpu.VMEM((_ER, _C), jnp.float32),
                pltpu.VMEM((_ER, _C), jnp.float32),
                pltpu.VMEM_SHARED((_N_LEVELS * _T,), jnp.int32),
            ]
            + [pltpu.SemaphoreType.DMA for _ in range(4 + 4 * _N_LEVELS)]
        ),
    )
    return f(px, py, pz, tabf)


def _mlp_body(enc_ref, pt_ref, w1_ref, w2_ref, o_ref):
    x = enc_ref[...]
    xb = x.astype(jnp.bfloat16)
    h = lax.dot_general(w1_ref[...].astype(jnp.bfloat16), xb,
                        (((0,), (0,)), ((), ())),
                        preferred_element_type=jnp.float32)
    h = jnp.maximum(h, 0.0).astype(jnp.bfloat16)
    d = lax.dot_general(w2_ref[...].astype(jnp.bfloat16), h,
                        (((0,), (0,)), ((), ())),
                        preferred_element_type=jnp.float32)
    p = pt_ref[...]
    ok = jnp.all((p > 0.0) & (p < 1.0), axis=0, keepdims=True)
    o_ref[...] = jnp.where(ok, jnp.exp(d - 1.0), jnp.float32(0.0))


def _mlp(enc, pt, w1, w2):
    n = enc.shape[1]
    nb = 16384
    return pl.pallas_call(
        _mlp_body,
        grid=(n // nb,),
        in_specs=[
            pl.BlockSpec((_ER, nb), lambda i: (0, i)),
            pl.BlockSpec((3, nb), lambda i: (0, i)),
            pl.BlockSpec((_ER, 64), lambda i: (0, 0)),
            pl.BlockSpec((64, 1), lambda i: (0, 0)),
        ],
        out_specs=pl.BlockSpec((1, nb), lambda i: (0, i)),
        out_shape=jax.ShapeDtypeStruct((1, n), jnp.float32),
    )(enc, pt, w1, w2)


def kernel(positions, table, W1, W2):
    n = positions.shape[0]
    tabf = jax.lax.bitcast_convert_type(
        table.reshape(_N_LEVELS * _T, 2).astype(jnp.bfloat16), jnp.int32)
    pt = positions.T
    px, py, pz = pt[0], pt[1], pt[2]
    h = n // 2
    outs = []
    for i in range(2):
        sl = slice(i * h, (i + 1) * h)
        enc = _encode(px[sl], py[sl], pz[sl], tabf)
        outs.append(_mlp(enc, pt[:, sl], W1, W2))
    out = jnp.concatenate(outs, axis=1)
    return out.reshape(n, 1)
